# baseline (speedup 1.0000x reference)
import functools

import jax
import jax.numpy as jnp
from jax import lax
from jax.experimental import pallas as pl
from jax.experimental.pallas import tpu as pltpu
from jax.experimental.pallas import tpu_sc as plsc

N = 10000
E = 160000
D_IN = 256
D_HID = 128
D_OUT = 64

NDEG = 10240
NC = 2
NS = 16
CH = 128
EC = E // NC
CPC = EC // CH
NCHT = CPC // NS
IDXL = (NCHT + 1) * CH
RPTD = NDEG // NS
RPT = 624
RPT_LAST = 640
NB = 2

_mesh = plsc.VectorSubcoreMesh(core_axis_name="c", subcore_axis_name="s")


@functools.partial(
    pl.kernel,
    mesh=_mesh,
    out_type=jax.ShapeDtypeStruct((NC * NDEG,), jnp.float32),
    scratch_types=[
        pltpu.VMEM((IDXL,), jnp.int32),
        pltpu.VMEM((CH,), jnp.float32),
        pltpu.VMEM((RPTD,), jnp.float32),
        pltpu.VMEM_SHARED((NDEG,), jnp.float32),
        pltpu.SemaphoreType.DMA,
        pltpu.SemaphoreType.DMA,
    ],
)
def _deg_kernel(ei_hbm, out_hbm, idx, ones_v, zero_v, acc, d0, d1):
    c = lax.axis_index("c")
    s = lax.axis_index("s")
    e0 = (c * CPC + s * NCHT) * CH
    rbase = s * RPTD
    idx_cp = pltpu.async_copy(ei_hbm.at[1, pl.ds(e0, NCHT * CH)],
                              idx.at[pl.ds(0, NCHT * CH)], d0)
    for i in range(RPTD // 16):
        zero_v[pl.ds(i * 16, 16)] = jnp.zeros((16,), jnp.float32)
    for i in range(CH // 16):
        ones_v[pl.ds(i * 16, 16)] = jnp.ones((16,), jnp.float32)
    pltpu.sync_copy(zero_v, acc.at[pl.ds(rbase, RPTD)])
    idx_cp.wait()

    @pl.when(s == 0)
    def _():
        pltpu.sync_copy(ei_hbm.at[1, pl.ds((c * CPC + NS * NCHT) * CH, CH)],
                        idx.at[pl.ds(NCHT * CH, CH)])

    plsc.subcore_barrier()

    def sstart(ci, sem):
        pltpu.async_copy(ones_v, acc.at[idx.at[pl.ds(ci * CH, CH)]],
                         sem, add=True)

    def swait(sem):
        pltpu.make_async_copy(ones_v, acc.at[idx.at[pl.ds(0, CH)]],
                              sem).wait()

    sstart(0, d0)
    sstart(1, d1)

    def body(j, carry):
        swait(d0)
        sstart(2 * j + 2, d0)
        swait(d1)
        sstart(2 * j + 3, d1)
        return carry

    lax.fori_loop(0, NCHT // 2 - 1, body, 0)
    swait(d0)
    sstart(NCHT - 1, d0)

    @pl.when(s == 0)
    def _():
        swait(d1)
        sstart(NCHT, d1)
        swait(d1)

    @pl.when(s != 0)
    def _():
        swait(d1)

    swait(d0)
    plsc.subcore_barrier()
    pltpu.sync_copy(acc.at[pl.ds(rbase, RPTD)],
                    out_hbm.at[pl.ds(c * NDEG + rbase, RPTD)])


@functools.partial(
    pl.kernel,
    mesh=_mesh,
    out_type=jax.ShapeDtypeStruct((NC * N, D_HID), jnp.float32),
    scratch_types=[
        pltpu.VMEM((IDXL,), jnp.int32),
        pltpu.VMEM((IDXL,), jnp.int32),
        pltpu.VMEM((8, D_HID), jnp.float32),
        pltpu.VMEM((CH, D_HID), jnp.float32),
        pltpu.VMEM((CH, D_HID), jnp.float32),
        pltpu.VMEM_SHARED((N, D_HID), jnp.float32),
        pltpu.SemaphoreType.DMA,
        pltpu.SemaphoreType.DMA,
        pltpu.SemaphoreType.DMA,
        pltpu.SemaphoreType.DMA,
    ],
)
def _scatter_kernel(g_hbm, ei_hbm, out_hbm, idxs, idxd, zb,
                    rows0, rows1, acc, sg0, sg1, ss0, ss1):
    c = lax.axis_index("c")
    s = lax.axis_index("s")
    e0 = (c * CPC + s * NCHT) * CH
    rbase = s * RPT
    icp0 = pltpu.async_copy(ei_hbm.at[0, pl.ds(e0, NCHT * CH)],
                            idxs.at[pl.ds(0, NCHT * CH)], sg0)
    icp1 = pltpu.async_copy(ei_hbm.at[1, pl.ds(e0, NCHT * CH)],
                            idxd.at[pl.ds(0, NCHT * CH)], sg1)
    for i in range(8):
        for j in range(D_HID // 16):
            zb[i, pl.ds(j * 16, 16)] = jnp.zeros((16,), jnp.float32)
    icp0.wait()
    icp1.wait()

    @pl.when(s == 0)
    def _():
        ex0 = (c * CPC + NS * NCHT) * CH
        pltpu.sync_copy(ei_hbm.at[0, pl.ds(ex0, CH)],
                        idxs.at[pl.ds(NCHT * CH, CH)])
        pltpu.sync_copy(ei_hbm.at[1, pl.ds(ex0, CH)],
                        idxd.at[pl.ds(NCHT * CH, CH)])

    rows = (rows0, rows1)
    sg = (sg0, sg1)
    ss = (ss0, ss1)

    def gstart(b, ci):
        pltpu.async_copy(g_hbm.at[idxs.at[pl.ds(ci * CH, CH)]], rows[b],
                         sg[b])

    def gwait(b):
        pltpu.make_async_copy(g_hbm.at[idxs.at[pl.ds(0, CH)]], rows[b],
                              sg[b]).wait()

    def sstart(b, ci):
        pltpu.async_copy(rows[b], acc.at[idxd.at[pl.ds(ci * CH, CH)]],
                         ss[b], add=True)

    def swait(b):
        pltpu.make_async_copy(rows[b], acc.at[idxd.at[pl.ds(0, CH)]],
                              ss[b]).wait()

    for b in range(NB):
        gstart(b, b)

    pltpu.sync_copy(zb, acc.at[pl.ds(rbase, 8)])

    @pl.when(s < NS - 1)
    def _():
        k = 8
        while 2 * k <= RPT:
            pltpu.sync_copy(acc.at[pl.ds(rbase, k)],
                            acc.at[pl.ds(rbase + k, k)])
            k *= 2
        if k < RPT:
            pltpu.sync_copy(acc.at[pl.ds(rbase, RPT - k)],
                            acc.at[pl.ds(rbase + k, RPT - k)])

    @pl.when(s == NS - 1)
    def _():
        k = 8
        while 2 * k <= RPT_LAST:
            pltpu.sync_copy(acc.at[pl.ds(rbase, k)],
                            acc.at[pl.ds(rbase + k, k)])
            k *= 2
        if k < RPT_LAST:
            pltpu.sync_copy(acc.at[pl.ds(rbase, RPT_LAST - k)],
                            acc.at[pl.ds(rbase + k, RPT_LAST - k)])

    plsc.subcore_barrier()

    def body(j, carry):
        base = NB * j
        for b in range(NB):
            gwait(b)
            sstart(b, base + b)

        @pl.when(j < (NCHT - 1) // NB - 1)
        def _():
            for b in range(NB):
                swait(b)
                gstart(b, base + NB + b)

        return carry

    lax.fori_loop(0, (NCHT - 1) // NB, body, 0)
    swait(0)
    gstart(0, NCHT - 1)
    gwait(0)
    sstart(0, NCHT - 1)

    @pl.when(s == 0)
    def _():
        swait(1)
        gstart(1, NCHT)
        gwait(1)
        sstart(1, NCHT)
        swait(1)

    @pl.when(s != 0)
    def _():
        swait(1)

    swait(0)
    plsc.subcore_barrier()

    @pl.when(s < NS - 1)
    def _():
        pltpu.sync_copy(acc.at[pl.ds(rbase, RPT)],
                        out_hbm.at[pl.ds(c * N + rbase, RPT)])

    @pl.when(s == NS - 1)
    def _():
        pltpu.sync_copy(acc.at[pl.ds(rbase, RPT_LAST)],
                        out_hbm.at[pl.ds(c * N + rbase, RPT_LAST)])


BN = 1024


def _mm1_body(x_ref, w_ref, p_ref, g_ref, dinv_ref):
    d = lax.rsqrt(1.0 + p_ref[0] + p_ref[1])[:, None]
    h = jnp.dot(x_ref[...], w_ref[...], preferred_element_type=jnp.float32)
    g_ref[...] = h * d
    dinv_ref[...] = d


def _mm2_body(s_ref, g1_ref, dinv_ref, b1_ref, wmu_ref, wls_ref, g2_ref):
    agg = s_ref[0] + s_ref[1] + g1_ref[...]
    h = jnp.maximum(dinv_ref[...] * agg + b1_ref[...], 0.0)
    dinv = dinv_ref[...]
    g2_ref[:, :D_OUT] = dinv * jnp.dot(
        h, wmu_ref[...], preferred_element_type=jnp.float32)
    g2_ref[:, D_OUT:] = dinv * jnp.dot(
        h, wls_ref[...], preferred_element_type=jnp.float32)


def _fin_body(t_ref, g2_ref, dinv_ref, bmu_ref, bls_ref, mu_ref, ls_ref):
    agg = t_ref[0] + t_ref[1] + g2_ref[...]
    o = dinv_ref[...] * agg
    mu_ref[...] = o[:, :D_OUT] + bmu_ref[...]
    ls_ref[...] = o[:, D_OUT:] + bls_ref[...]


def _col_spec():
    return pl.BlockSpec((BN, 1), lambda i: (i, 0))


def _row_spec(d):
    return pl.BlockSpec((BN, d), lambda i: (i, 0))


def _deg_spec():
    return pl.BlockSpec((NC, BN), lambda i: (0, i))


def _full_spec(r, d):
    return pl.BlockSpec((r, d), lambda i: (0, 0))


def _vec_spec(d):
    return pl.BlockSpec((d,), lambda i: (0,))


def _pair_spec(d):
    return pl.BlockSpec((NC, BN, d), lambda i: (0, i, 0))


def kernel(x, edge_index, W1, b1, Wmu, bmu, Wls, bls):
    degp = _deg_kernel(edge_index).reshape(NC, NDEG)

    grid = pl.cdiv(N, BN)

    g1, dinv = pl.pallas_call(
        _mm1_body,
        grid=(grid,),
        in_specs=[_row_spec(D_IN), _full_spec(D_IN, D_HID), _deg_spec()],
        out_specs=[_row_spec(D_HID), _col_spec()],
        out_shape=[jax.ShapeDtypeStruct((N, D_HID), jnp.float32),
                   jax.ShapeDtypeStruct((N, 1), jnp.float32)],
    )(x, W1, degp)

    s_pair = _scatter_kernel(g1, edge_index).reshape(NC, N, D_HID)

    g2 = pl.pallas_call(
        _mm2_body,
        grid=(grid,),
        in_specs=[_pair_spec(D_HID), _row_spec(D_HID), _col_spec(),
                  _vec_spec(D_HID), _full_spec(D_HID, D_OUT),
                  _full_spec(D_HID, D_OUT)],
        out_specs=_row_spec(D_HID),
        out_shape=jax.ShapeDtypeStruct((N, D_HID), jnp.float32),
    )(s_pair, g1, dinv, b1, Wmu, Wls)

    t_pair = _scatter_kernel(g2, edge_index).reshape(NC, N, D_HID)

    mu, logstd = pl.pallas_call(
        _fin_body,
        grid=(grid,),
        in_specs=[_pair_spec(D_HID), _row_spec(D_HID), _col_spec(),
                  _vec_spec(D_OUT), _vec_spec(D_OUT)],
        out_specs=[_row_spec(D_OUT), _row_spec(D_OUT)],
        out_shape=[jax.ShapeDtypeStruct((N, D_OUT), jnp.float32),
                   jax.ShapeDtypeStruct((N, D_OUT), jnp.float32)],
    )(t_pair, g2, dinv, bmu, bls)

    return (mu, logstd)


# speedup vs baseline: 3.5538x; 3.5538x over previous
import functools

import jax
import jax.numpy as jnp
from jax import lax
from jax.experimental import pallas as pl
from jax.experimental.pallas import tpu as pltpu
from jax.experimental.pallas import tpu_sc as plsc

N = 10000
E = 160000
D_IN = 256
D_HID = 128
D_OUT = 64

NDEG = 10240
NC = 2
NS = 16
CH = 128
EC = E // NC
CPC = EC // CH
NCHT = CPC // NS
IDXL = (NCHT + 1) * CH
RPTD = NDEG // NS
RPT = 624
RPT_LAST = 640
NB = 2

_mesh = plsc.VectorSubcoreMesh(core_axis_name="c", subcore_axis_name="s")


@functools.partial(
    pl.kernel,
    mesh=_mesh,
    out_type=jax.ShapeDtypeStruct((NC * NDEG,), jnp.float32),
    scratch_types=[
        pltpu.VMEM((IDXL,), jnp.int32),
        pltpu.VMEM((CH,), jnp.float32),
        pltpu.VMEM((RPTD,), jnp.float32),
        pltpu.VMEM_SHARED((NDEG,), jnp.float32),
        pltpu.SemaphoreType.DMA,
        pltpu.SemaphoreType.DMA,
    ],
)
def _deg_kernel(ei_hbm, out_hbm, idx, ones_v, zero_v, acc, d0, d1):
    c = lax.axis_index("c")
    s = lax.axis_index("s")
    e0 = (c * CPC + s * NCHT) * CH
    rbase = s * RPTD
    idx_cp = pltpu.async_copy(ei_hbm.at[1, pl.ds(e0, NCHT * CH)],
                              idx.at[pl.ds(0, NCHT * CH)], d0)
    for i in range(RPTD // 16):
        zero_v[pl.ds(i * 16, 16)] = jnp.zeros((16,), jnp.float32)
    for i in range(CH // 16):
        ones_v[pl.ds(i * 16, 16)] = jnp.ones((16,), jnp.float32)
    pltpu.sync_copy(zero_v, acc.at[pl.ds(rbase, RPTD)])
    idx_cp.wait()

    @pl.when(s == 0)
    def _():
        pltpu.sync_copy(ei_hbm.at[1, pl.ds((c * CPC + NS * NCHT) * CH, CH)],
                        idx.at[pl.ds(NCHT * CH, CH)])

    plsc.subcore_barrier()

    def sstart(ci, sem):
        pltpu.async_copy(ones_v, acc.at[idx.at[pl.ds(ci * CH, CH)]],
                         sem, add=True)

    def swait(sem):
        pltpu.make_async_copy(ones_v, acc.at[idx.at[pl.ds(0, CH)]],
                              sem).wait()

    sstart(0, d0)
    sstart(1, d1)

    def body(j, carry):
        swait(d0)
        sstart(2 * j + 2, d0)
        swait(d1)
        sstart(2 * j + 3, d1)
        return carry

    lax.fori_loop(0, NCHT // 2 - 1, body, 0)
    swait(d0)
    sstart(NCHT - 1, d0)

    @pl.when(s == 0)
    def _():
        swait(d1)
        sstart(NCHT, d1)
        swait(d1)

    @pl.when(s != 0)
    def _():
        swait(d1)

    swait(d0)
    plsc.subcore_barrier()
    pltpu.sync_copy(acc.at[pl.ds(rbase, RPTD)],
                    out_hbm.at[pl.ds(c * NDEG + rbase, RPTD)])


@functools.partial(
    pl.kernel,
    mesh=_mesh,
    out_type=jax.ShapeDtypeStruct((NC * N, D_HID), jnp.float32),
    scratch_types=[
        pltpu.VMEM((IDXL,), jnp.int32),
        pltpu.VMEM((IDXL,), jnp.int32),
        pltpu.VMEM((CH, D_HID), jnp.float32),
        pltpu.VMEM((CH, D_HID), jnp.float32),
        pltpu.VMEM_SHARED((N, D_HID), jnp.float32),
        pltpu.SemaphoreType.DMA,
        pltpu.SemaphoreType.DMA,
        pltpu.SemaphoreType.DMA,
        pltpu.SemaphoreType.DMA,
    ],
)
def _scatter_kernel(g_hbm, ei_hbm, out_hbm, idxs, idxd,
                    rows0, rows1, acc, sg0, sg1, ss0, ss1):
    c = lax.axis_index("c")
    s = lax.axis_index("s")
    e0 = (c * CPC + s * NCHT) * CH
    rbase = s * RPT
    icp0 = pltpu.async_copy(ei_hbm.at[0, pl.ds(e0, NCHT * CH)],
                            idxs.at[pl.ds(0, NCHT * CH)], sg0)
    icp1 = pltpu.async_copy(ei_hbm.at[1, pl.ds(e0, NCHT * CH)],
                            idxd.at[pl.ds(0, NCHT * CH)], sg1)
    icp0.wait()
    icp1.wait()

    @pl.when(s == 0)
    def _():
        ex0 = (c * CPC + NS * NCHT) * CH
        pltpu.sync_copy(ei_hbm.at[0, pl.ds(ex0, CH)],
                        idxs.at[pl.ds(NCHT * CH, CH)])
        pltpu.sync_copy(ei_hbm.at[1, pl.ds(ex0, CH)],
                        idxd.at[pl.ds(NCHT * CH, CH)])

    rows = (rows0, rows1)
    sg = (sg0, sg1)
    ss = (ss0, ss1)

    def gstart(b, ci):
        pltpu.async_copy(g_hbm.at[idxs.at[pl.ds(ci * CH, CH)]], rows[b],
                         sg[b])

    def gwait(b):
        pltpu.make_async_copy(g_hbm.at[idxs.at[pl.ds(0, CH)]], rows[b],
                              sg[b]).wait()

    def sstart(b, ci):
        pltpu.async_copy(rows[b], acc.at[idxd.at[pl.ds(ci * CH, CH)]],
                         ss[b], add=True)

    def swait(b):
        pltpu.make_async_copy(rows[b], acc.at[idxd.at[pl.ds(0, CH)]],
                              ss[b]).wait()

    for b in range(NB):
        gstart(b, b)

    @pl.when(s < NS - 1)
    def _():
        pltpu.sync_copy(g_hbm.at[pl.ds(rbase, RPT)],
                        acc.at[pl.ds(rbase, RPT)])

    @pl.when(s == NS - 1)
    def _():
        pltpu.sync_copy(g_hbm.at[pl.ds(rbase, RPT_LAST)],
                        acc.at[pl.ds(rbase, RPT_LAST)])

    plsc.subcore_barrier()

    def body(j, carry):
        base = NB * j
        for b in range(NB):
            gwait(b)
            sstart(b, base + b)

        @pl.when(j < (NCHT - 1) // NB - 1)
        def _():
            for b in range(NB):
                swait(b)
                gstart(b, base + NB + b)

        return carry

    lax.fori_loop(0, (NCHT - 1) // NB, body, 0)
    swait(0)
    gstart(0, NCHT - 1)
    gwait(0)
    sstart(0, NCHT - 1)

    @pl.when(s == 0)
    def _():
        swait(1)
        gstart(1, NCHT)
        gwait(1)
        sstart(1, NCHT)
        swait(1)

    @pl.when(s != 0)
    def _():
        swait(1)

    swait(0)
    plsc.subcore_barrier()

    @pl.when(s < NS - 1)
    def _():
        pltpu.sync_copy(acc.at[pl.ds(rbase, RPT)],
                        out_hbm.at[pl.ds(c * N + rbase, RPT)])

    @pl.when(s == NS - 1)
    def _():
        pltpu.sync_copy(acc.at[pl.ds(rbase, RPT_LAST)],
                        out_hbm.at[pl.ds(c * N + rbase, RPT_LAST)])


BN = 1024


def _mm1_body(x_ref, w_ref, p_ref, g_ref, dinv_ref):
    d = lax.rsqrt(1.0 + p_ref[0] + p_ref[1])[:, None]
    h = jnp.dot(x_ref[...], w_ref[...], preferred_element_type=jnp.float32)
    g_ref[...] = h * d
    dinv_ref[...] = d


def _mm2_body(s_ref, g1_ref, dinv_ref, b1_ref, wmu_ref, wls_ref, g2_ref):
    agg = s_ref[0] + s_ref[1] - g1_ref[...]
    h = jnp.maximum(dinv_ref[...] * agg + b1_ref[...], 0.0)
    dinv = dinv_ref[...]
    g2_ref[:, :D_OUT] = dinv * jnp.dot(
        h, wmu_ref[...], preferred_element_type=jnp.float32)
    g2_ref[:, D_OUT:] = dinv * jnp.dot(
        h, wls_ref[...], preferred_element_type=jnp.float32)


def _fin_body(t_ref, g2_ref, dinv_ref, bmu_ref, bls_ref, mu_ref, ls_ref):
    agg = t_ref[0] + t_ref[1] - g2_ref[...]
    o = dinv_ref[...] * agg
    mu_ref[...] = o[:, :D_OUT] + bmu_ref[...]
    ls_ref[...] = o[:, D_OUT:] + bls_ref[...]


def _col_spec():
    return pl.BlockSpec((BN, 1), lambda i: (i, 0))


def _row_spec(d):
    return pl.BlockSpec((BN, d), lambda i: (i, 0))


def _deg_spec():
    return pl.BlockSpec((NC, BN), lambda i: (0, i))


def _full_spec(r, d):
    return pl.BlockSpec((r, d), lambda i: (0, 0))


def _vec_spec(d):
    return pl.BlockSpec((d,), lambda i: (0,))


def _pair_spec(d):
    return pl.BlockSpec((NC, BN, d), lambda i: (0, i, 0))


def kernel(x, edge_index, W1, b1, Wmu, bmu, Wls, bls):
    degp = _deg_kernel(edge_index).reshape(NC, NDEG)

    grid = pl.cdiv(N, BN)

    g1, dinv = pl.pallas_call(
        _mm1_body,
        grid=(grid,),
        in_specs=[_row_spec(D_IN), _full_spec(D_IN, D_HID), _deg_spec()],
        out_specs=[_row_spec(D_HID), _col_spec()],
        out_shape=[jax.ShapeDtypeStruct((N, D_HID), jnp.float32),
                   jax.ShapeDtypeStruct((N, 1), jnp.float32)],
    )(x, W1, degp)

    s_pair = _scatter_kernel(g1, edge_index).reshape(NC, N, D_HID)

    g2 = pl.pallas_call(
        _mm2_body,
        grid=(grid,),
        in_specs=[_pair_spec(D_HID), _row_spec(D_HID), _col_spec(),
                  _vec_spec(D_HID), _full_spec(D_HID, D_OUT),
                  _full_spec(D_HID, D_OUT)],
        out_specs=_row_spec(D_HID),
        out_shape=jax.ShapeDtypeStruct((N, D_HID), jnp.float32),
    )(s_pair, g1, dinv, b1, Wmu, Wls)

    t_pair = _scatter_kernel(g2, edge_index).reshape(NC, N, D_HID)

    mu, logstd = pl.pallas_call(
        _fin_body,
        grid=(grid,),
        in_specs=[_pair_spec(D_HID), _row_spec(D_HID), _col_spec(),
                  _vec_spec(D_OUT), _vec_spec(D_OUT)],
        out_specs=[_row_spec(D_OUT), _row_spec(D_OUT)],
        out_shape=[jax.ShapeDtypeStruct((N, D_OUT), jnp.float32),
                   jax.ShapeDtypeStruct((N, D_OUT), jnp.float32)],
    )(t_pair, g2, dinv, bmu, bls)

    return (mu, logstd)


# speedup vs baseline: 4.0833x; 1.1490x over previous
import functools

import jax
import jax.numpy as jnp
from jax import lax
from jax.experimental import pallas as pl
from jax.experimental.pallas import tpu as pltpu
from jax.experimental.pallas import tpu_sc as plsc

N = 10000
E = 160000
D_IN = 256
D_HID = 128
D_OUT = 64

NDEG = 10240
NC = 2
NS = 16
CH = 128
EC = E // NC
CPC = EC // CH
NCHT = CPC // NS
IDXL = (NCHT + 1) * CH
RPTD = NDEG // NS
RPT = 624
RPT_LAST = 640
NB = 2

CH2 = 64
CPC2 = EC // CH2
NCHT2 = CPC2 // NS
IDXL2 = (NCHT2 + 1) * CH2
NB2 = 4

_mesh = plsc.VectorSubcoreMesh(core_axis_name="c", subcore_axis_name="s")


@functools.partial(
    pl.kernel,
    mesh=_mesh,
    out_type=jax.ShapeDtypeStruct((NC * NDEG,), jnp.float32),
    scratch_types=[
        pltpu.VMEM((IDXL,), jnp.int32),
        pltpu.VMEM((CH,), jnp.float32),
        pltpu.VMEM((RPTD,), jnp.float32),
        pltpu.VMEM_SHARED((NDEG,), jnp.float32),
        pltpu.SemaphoreType.DMA,
        pltpu.SemaphoreType.DMA,
    ],
)
def _deg_kernel(ei_hbm, out_hbm, idx, ones_v, zero_v, acc, d0, d1):
    c = lax.axis_index("c")
    s = lax.axis_index("s")
    e0 = (c * CPC + s * NCHT) * CH
    rbase = s * RPTD
    idx_cp = pltpu.async_copy(ei_hbm.at[1, pl.ds(e0, NCHT * CH)],
                              idx.at[pl.ds(0, NCHT * CH)], d0)
    for i in range(RPTD // 16):
        zero_v[pl.ds(i * 16, 16)] = jnp.zeros((16,), jnp.float32)
    for i in range(CH // 16):
        ones_v[pl.ds(i * 16, 16)] = jnp.ones((16,), jnp.float32)
    pltpu.sync_copy(zero_v, acc.at[pl.ds(rbase, RPTD)])
    idx_cp.wait()

    @pl.when(s == 0)
    def _():
        pltpu.sync_copy(ei_hbm.at[1, pl.ds((c * CPC + NS * NCHT) * CH, CH)],
                        idx.at[pl.ds(NCHT * CH, CH)])

    plsc.subcore_barrier()

    def sstart(ci, sem):
        pltpu.async_copy(ones_v, acc.at[idx.at[pl.ds(ci * CH, CH)]],
                         sem, add=True)

    def swait(sem):
        pltpu.make_async_copy(ones_v, acc.at[idx.at[pl.ds(0, CH)]],
                              sem).wait()

    sstart(0, d0)
    sstart(1, d1)

    def body(j, carry):
        swait(d0)
        sstart(2 * j + 2, d0)
        swait(d1)
        sstart(2 * j + 3, d1)
        return carry

    lax.fori_loop(0, NCHT // 2 - 1, body, 0)
    swait(d0)
    sstart(NCHT - 1, d0)

    @pl.when(s == 0)
    def _():
        swait(d1)
        sstart(NCHT, d1)
        swait(d1)

    @pl.when(s != 0)
    def _():
        swait(d1)

    swait(d0)
    plsc.subcore_barrier()
    pltpu.sync_copy(acc.at[pl.ds(rbase, RPTD)],
                    out_hbm.at[pl.ds(c * NDEG + rbase, RPTD)])


@functools.partial(
    pl.kernel,
    mesh=_mesh,
    out_type=jax.ShapeDtypeStruct((NC * N, D_HID), jnp.float32),
    scratch_types=[
        pltpu.VMEM((IDXL2,), jnp.int32),
        pltpu.VMEM((IDXL2,), jnp.int32),
        pltpu.VMEM((CH2, D_HID), jnp.float32),
        pltpu.VMEM((CH2, D_HID), jnp.float32),
        pltpu.VMEM((CH2, D_HID), jnp.float32),
        pltpu.VMEM((CH2, D_HID), jnp.float32),
        pltpu.VMEM_SHARED((N, D_HID), jnp.float32),
        pltpu.SemaphoreType.DMA,
        pltpu.SemaphoreType.DMA,
        pltpu.SemaphoreType.DMA,
        pltpu.SemaphoreType.DMA,
        pltpu.SemaphoreType.DMA,
        pltpu.SemaphoreType.DMA,
        pltpu.SemaphoreType.DMA,
        pltpu.SemaphoreType.DMA,
    ],
)
def _scatter_kernel(g_hbm, ei_hbm, out_hbm, idxs, idxd,
                    rows0, rows1, rows2, rows3, acc,
                    sg0, sg1, sg2, sg3, ss0, ss1, ss2, ss3):
    c = lax.axis_index("c")
    s = lax.axis_index("s")
    e0 = (c * CPC2 + s * NCHT2) * CH2
    rbase = s * RPT
    icp0 = pltpu.async_copy(ei_hbm.at[0, pl.ds(e0, NCHT2 * CH2)],
                            idxs.at[pl.ds(0, NCHT2 * CH2)], sg0)
    icp1 = pltpu.async_copy(ei_hbm.at[1, pl.ds(e0, NCHT2 * CH2)],
                            idxd.at[pl.ds(0, NCHT2 * CH2)], sg1)
    icp0.wait()
    icp1.wait()

    @pl.when(s < 2)
    def _():
        ex0 = (c * CPC2 + NS * NCHT2 + s) * CH2
        pltpu.sync_copy(ei_hbm.at[0, pl.ds(ex0, CH2)],
                        idxs.at[pl.ds(NCHT2 * CH2, CH2)])
        pltpu.sync_copy(ei_hbm.at[1, pl.ds(ex0, CH2)],
                        idxd.at[pl.ds(NCHT2 * CH2, CH2)])

    rows = (rows0, rows1, rows2, rows3)
    sg = (sg0, sg1, sg2, sg3)
    ss = (ss0, ss1, ss2, ss3)

    def gstart(b, ci):
        pltpu.async_copy(g_hbm.at[idxs.at[pl.ds(ci * CH2, CH2)]], rows[b],
                         sg[b])

    def gwait(b):
        pltpu.make_async_copy(g_hbm.at[idxs.at[pl.ds(0, CH2)]], rows[b],
                              sg[b]).wait()

    def sstart(b, ci):
        pltpu.async_copy(rows[b], acc.at[idxd.at[pl.ds(ci * CH2, CH2)]],
                         ss[b], add=True)

    def swait(b):
        pltpu.make_async_copy(rows[b], acc.at[idxd.at[pl.ds(0, CH2)]],
                              ss[b]).wait()

    for b in range(NB2):
        gstart(b, b)

    @pl.when(s < NS - 1)
    def _():
        pltpu.sync_copy(g_hbm.at[pl.ds(rbase, RPT)],
                        acc.at[pl.ds(rbase, RPT)])

    @pl.when(s == NS - 1)
    def _():
        pltpu.sync_copy(g_hbm.at[pl.ds(rbase, RPT_LAST)],
                        acc.at[pl.ds(rbase, RPT_LAST)])

    plsc.subcore_barrier()

    NQ = (NCHT2 - 2) // NB2

    def body(j, carry):
        base = NB2 * j
        for b in range(NB2):
            gwait(b)
            sstart(b, base + b)

        @pl.when(j < NQ - 1)
        def _():
            for b in range(NB2):
                swait(b)
                gstart(b, base + NB2 + b)

        return carry

    lax.fori_loop(0, NQ, body, 0)
    swait(0)
    gstart(0, NCHT2 - 2)
    swait(1)
    gstart(1, NCHT2 - 1)

    @pl.when(s < 2)
    def _():
        swait(2)
        gstart(2, NCHT2)
        gwait(2)
        sstart(2, NCHT2)
        swait(2)

    @pl.when(s >= 2)
    def _():
        swait(2)

    gwait(0)
    sstart(0, NCHT2 - 2)
    gwait(1)
    sstart(1, NCHT2 - 1)
    swait(3)
    swait(0)
    swait(1)
    plsc.subcore_barrier()

    @pl.when(s < NS - 1)
    def _():
        pltpu.sync_copy(acc.at[pl.ds(rbase, RPT)],
                        out_hbm.at[pl.ds(c * N + rbase, RPT)])

    @pl.when(s == NS - 1)
    def _():
        pltpu.sync_copy(acc.at[pl.ds(rbase, RPT_LAST)],
                        out_hbm.at[pl.ds(c * N + rbase, RPT_LAST)])


BN = 1024


def _mm1_body(x_ref, w_ref, p_ref, g_ref, dinv_ref):
    d = lax.rsqrt(1.0 + p_ref[0] + p_ref[1])[:, None]
    h = jnp.dot(x_ref[...], w_ref[...], preferred_element_type=jnp.float32)
    g_ref[...] = h * d
    dinv_ref[...] = d


def _mm2_body(s_ref, g1_ref, dinv_ref, b1_ref, wmu_ref, wls_ref, g2_ref):
    agg = s_ref[0] + s_ref[1] - g1_ref[...]
    h = jnp.maximum(dinv_ref[...] * agg + b1_ref[...], 0.0)
    dinv = dinv_ref[...]
    g2_ref[:, :D_OUT] = dinv * jnp.dot(
        h, wmu_ref[...], preferred_element_type=jnp.float32)
    g2_ref[:, D_OUT:] = dinv * jnp.dot(
        h, wls_ref[...], preferred_element_type=jnp.float32)


def _fin_body(t_ref, g2_ref, dinv_ref, bmu_ref, bls_ref, mu_ref, ls_ref):
    agg = t_ref[0] + t_ref[1] - g2_ref[...]
    o = dinv_ref[...] * agg
    mu_ref[...] = o[:, :D_OUT] + bmu_ref[...]
    ls_ref[...] = o[:, D_OUT:] + bls_ref[...]


def _col_spec():
    return pl.BlockSpec((BN, 1), lambda i: (i, 0))


def _row_spec(d):
    return pl.BlockSpec((BN, d), lambda i: (i, 0))


def _deg_spec():
    return pl.BlockSpec((NC, BN), lambda i: (0, i))


def _full_spec(r, d):
    return pl.BlockSpec((r, d), lambda i: (0, 0))


def _vec_spec(d):
    return pl.BlockSpec((d,), lambda i: (0,))


def _pair_spec(d):
    return pl.BlockSpec((NC, BN, d), lambda i: (0, i, 0))


def kernel(x, edge_index, W1, b1, Wmu, bmu, Wls, bls):
    degp = _deg_kernel(edge_index).reshape(NC, NDEG)

    grid = pl.cdiv(N, BN)

    g1, dinv = pl.pallas_call(
        _mm1_body,
        grid=(grid,),
        in_specs=[_row_spec(D_IN), _full_spec(D_IN, D_HID), _deg_spec()],
        out_specs=[_row_spec(D_HID), _col_spec()],
        out_shape=[jax.ShapeDtypeStruct((N, D_HID), jnp.float32),
                   jax.ShapeDtypeStruct((N, 1), jnp.float32)],
    )(x, W1, degp)

    s_pair = _scatter_kernel(g1, edge_index).reshape(NC, N, D_HID)

    g2 = pl.pallas_call(
        _mm2_body,
        grid=(grid,),
        in_specs=[_pair_spec(D_HID), _row_spec(D_HID), _col_spec(),
                  _vec_spec(D_HID), _full_spec(D_HID, D_OUT),
                  _full_spec(D_HID, D_OUT)],
        out_specs=_row_spec(D_HID),
        out_shape=jax.ShapeDtypeStruct((N, D_HID), jnp.float32),
    )(s_pair, g1, dinv, b1, Wmu, Wls)

    t_pair = _scatter_kernel(g2, edge_index).reshape(NC, N, D_HID)

    mu, logstd = pl.pallas_call(
        _fin_body,
        grid=(grid,),
        in_specs=[_pair_spec(D_HID), _row_spec(D_HID), _col_spec(),
                  _vec_spec(D_OUT), _vec_spec(D_OUT)],
        out_specs=[_row_spec(D_OUT), _row_spec(D_OUT)],
        out_shape=[jax.ShapeDtypeStruct((N, D_OUT), jnp.float32),
                   jax.ShapeDtypeStruct((N, D_OUT), jnp.float32)],
    )(t_pair, g2, dinv, bmu, bls)

    return (mu, logstd)


# speedup vs baseline: 4.1431x; 1.0147x over previous
import functools

import jax
import jax.numpy as jnp
from jax import lax
from jax.experimental import pallas as pl
from jax.experimental.pallas import tpu as pltpu
from jax.experimental.pallas import tpu_sc as plsc

N = 10000
E = 160000
D_IN = 256
D_HID = 128
D_OUT = 64

NDEG = 10240
NC = 2
NS = 16
CH = 128
EC = E // NC
CPC = EC // CH
NCHT = CPC // NS
IDXL = (NCHT + 1) * CH
RPTD = NDEG // NS
RPT = 624
RPT_LAST = 640
NB = 2

CH2 = 32
CPC2 = EC // CH2
NCHT2 = CPC2 // NS
NXTRA = CPC2 - NS * NCHT2
IDXL2 = (NCHT2 + 1) * CH2
NB2 = 8
NQ2 = 19

_mesh = plsc.VectorSubcoreMesh(core_axis_name="c", subcore_axis_name="s")


@functools.partial(
    pl.kernel,
    mesh=_mesh,
    out_type=jax.ShapeDtypeStruct((NC * NDEG,), jnp.float32),
    scratch_types=[
        pltpu.VMEM((IDXL,), jnp.int32),
        pltpu.VMEM((CH,), jnp.float32),
        pltpu.VMEM((RPTD,), jnp.float32),
        pltpu.VMEM_SHARED((NDEG,), jnp.float32),
        pltpu.SemaphoreType.DMA,
        pltpu.SemaphoreType.DMA,
    ],
)
def _deg_kernel(ei_hbm, out_hbm, idx, ones_v, zero_v, acc, d0, d1):
    c = lax.axis_index("c")
    s = lax.axis_index("s")
    e0 = (c * CPC + s * NCHT) * CH
    rbase = s * RPTD
    idx_cp = pltpu.async_copy(ei_hbm.at[1, pl.ds(e0, NCHT * CH)],
                              idx.at[pl.ds(0, NCHT * CH)], d0)
    for i in range(RPTD // 16):
        zero_v[pl.ds(i * 16, 16)] = jnp.zeros((16,), jnp.float32)
    for i in range(CH // 16):
        ones_v[pl.ds(i * 16, 16)] = jnp.ones((16,), jnp.float32)
    pltpu.sync_copy(zero_v, acc.at[pl.ds(rbase, RPTD)])
    idx_cp.wait()

    @pl.when(s == 0)
    def _():
        pltpu.sync_copy(ei_hbm.at[1, pl.ds((c * CPC + NS * NCHT) * CH, CH)],
                        idx.at[pl.ds(NCHT * CH, CH)])

    plsc.subcore_barrier()

    def sstart(ci, sem):
        pltpu.async_copy(ones_v, acc.at[idx.at[pl.ds(ci * CH, CH)]],
                         sem, add=True)

    def swait(sem):
        pltpu.make_async_copy(ones_v, acc.at[idx.at[pl.ds(0, CH)]],
                              sem).wait()

    sstart(0, d0)
    sstart(1, d1)

    def body(j, carry):
        swait(d0)
        sstart(2 * j + 2, d0)
        swait(d1)
        sstart(2 * j + 3, d1)
        return carry

    lax.fori_loop(0, NCHT // 2 - 1, body, 0)
    swait(d0)
    sstart(NCHT - 1, d0)

    @pl.when(s == 0)
    def _():
        swait(d1)
        sstart(NCHT, d1)
        swait(d1)

    @pl.when(s != 0)
    def _():
        swait(d1)

    swait(d0)
    plsc.subcore_barrier()
    pltpu.sync_copy(acc.at[pl.ds(rbase, RPTD)],
                    out_hbm.at[pl.ds(c * NDEG + rbase, RPTD)])


@functools.partial(
    pl.kernel,
    mesh=_mesh,
    out_type=jax.ShapeDtypeStruct((NC * N, D_HID), jnp.float32),
    scratch_types=[
        pltpu.VMEM((IDXL2,), jnp.int32),
        pltpu.VMEM((IDXL2,), jnp.int32),
    ] + [pltpu.VMEM((CH2, D_HID), jnp.float32)] * NB2 + [
        pltpu.VMEM_SHARED((N, D_HID), jnp.float32),
    ] + [pltpu.SemaphoreType.DMA] * (2 * NB2),
)
def _scatter_kernel(g_hbm, ei_hbm, out_hbm, idxs, idxd, *rest):
    rows = rest[:NB2]
    acc = rest[NB2]
    sg = rest[NB2 + 1:2 * NB2 + 1]
    ss = rest[2 * NB2 + 1:]
    sg0, sg1 = sg[0], sg[1]
    c = lax.axis_index("c")
    s = lax.axis_index("s")
    e0 = (c * CPC2 + s * NCHT2) * CH2
    rbase = s * RPT
    icp0 = pltpu.async_copy(ei_hbm.at[0, pl.ds(e0, NCHT2 * CH2)],
                            idxs.at[pl.ds(0, NCHT2 * CH2)], sg0)
    icp1 = pltpu.async_copy(ei_hbm.at[1, pl.ds(e0, NCHT2 * CH2)],
                            idxd.at[pl.ds(0, NCHT2 * CH2)], sg1)
    icp0.wait()
    icp1.wait()

    @pl.when(s < NXTRA)
    def _():
        ex0 = (c * CPC2 + NS * NCHT2 + s) * CH2
        pltpu.sync_copy(ei_hbm.at[0, pl.ds(ex0, CH2)],
                        idxs.at[pl.ds(NCHT2 * CH2, CH2)])
        pltpu.sync_copy(ei_hbm.at[1, pl.ds(ex0, CH2)],
                        idxd.at[pl.ds(NCHT2 * CH2, CH2)])

    def gstart(b, ci):
        pltpu.async_copy(g_hbm.at[idxs.at[pl.ds(ci * CH2, CH2)]], rows[b],
                         sg[b])

    def gwait(b):
        pltpu.make_async_copy(g_hbm.at[idxs.at[pl.ds(0, CH2)]], rows[b],
                              sg[b]).wait()

    def sstart(b, ci):
        pltpu.async_copy(rows[b], acc.at[idxd.at[pl.ds(ci * CH2, CH2)]],
                         ss[b], add=True)

    def swait(b):
        pltpu.make_async_copy(rows[b], acc.at[idxd.at[pl.ds(0, CH2)]],
                              ss[b]).wait()

    for b in range(NB2):
        gstart(b, b)

    @pl.when(s < NS - 1)
    def _():
        pltpu.sync_copy(g_hbm.at[pl.ds(rbase, RPT)],
                        acc.at[pl.ds(rbase, RPT)])

    @pl.when(s == NS - 1)
    def _():
        pltpu.sync_copy(g_hbm.at[pl.ds(rbase, RPT_LAST)],
                        acc.at[pl.ds(rbase, RPT_LAST)])

    plsc.subcore_barrier()

    NREM = NCHT2 - NB2 * NQ2

    def body(j, carry):
        base = NB2 * j
        for b in range(NB2):
            gwait(b)
            sstart(b, base + b)

        @pl.when(j < NQ2 - 1)
        def _():
            for b in range(NB2):
                swait(b)
                gstart(b, base + NB2 + b)

        return carry

    lax.fori_loop(0, NQ2, body, 0)
    for b in range(NREM):
        swait(b)
        gstart(b, NB2 * NQ2 + b)

    @pl.when(s < NXTRA)
    def _():
        swait(NREM)
        gstart(NREM, NCHT2)
        gwait(NREM)
        sstart(NREM, NCHT2)
        swait(NREM)

    @pl.when(s >= NXTRA)
    def _():
        swait(NREM)

    for b in range(NREM):
        gwait(b)
        sstart(b, NB2 * NQ2 + b)
    for b in range(NREM + 1, NB2):
        swait(b)
    for b in range(NREM):
        swait(b)
    plsc.subcore_barrier()

    @pl.when(s < NS - 1)
    def _():
        pltpu.sync_copy(acc.at[pl.ds(rbase, RPT)],
                        out_hbm.at[pl.ds(c * N + rbase, RPT)])

    @pl.when(s == NS - 1)
    def _():
        pltpu.sync_copy(acc.at[pl.ds(rbase, RPT_LAST)],
                        out_hbm.at[pl.ds(c * N + rbase, RPT_LAST)])


BN = 1024


def _mm1_body(x_ref, w_ref, p_ref, g_ref, dinv_ref):
    d = lax.rsqrt(1.0 + p_ref[0] + p_ref[1])[:, None]
    h = jnp.dot(x_ref[...], w_ref[...], preferred_element_type=jnp.float32)
    g_ref[...] = h * d
    dinv_ref[...] = d


def _mm2_body(s_ref, g1_ref, dinv_ref, b1_ref, wmu_ref, wls_ref, g2_ref):
    agg = s_ref[0] + s_ref[1] - g1_ref[...]
    h = jnp.maximum(dinv_ref[...] * agg + b1_ref[...], 0.0)
    dinv = dinv_ref[...]
    g2_ref[:, :D_OUT] = dinv * jnp.dot(
        h, wmu_ref[...], preferred_element_type=jnp.float32)
    g2_ref[:, D_OUT:] = dinv * jnp.dot(
        h, wls_ref[...], preferred_element_type=jnp.float32)


def _fin_body(t_ref, g2_ref, dinv_ref, bmu_ref, bls_ref, mu_ref, ls_ref):
    agg = t_ref[0] + t_ref[1] - g2_ref[...]
    o = dinv_ref[...] * agg
    mu_ref[...] = o[:, :D_OUT] + bmu_ref[...]
    ls_ref[...] = o[:, D_OUT:] + bls_ref[...]


def _col_spec():
    return pl.BlockSpec((BN, 1), lambda i: (i, 0))


def _row_spec(d):
    return pl.BlockSpec((BN, d), lambda i: (i, 0))


def _deg_spec():
    return pl.BlockSpec((NC, BN), lambda i: (0, i))


def _full_spec(r, d):
    return pl.BlockSpec((r, d), lambda i: (0, 0))


def _vec_spec(d):
    return pl.BlockSpec((d,), lambda i: (0,))


def _pair_spec(d):
    return pl.BlockSpec((NC, BN, d), lambda i: (0, i, 0))


def kernel(x, edge_index, W1, b1, Wmu, bmu, Wls, bls):
    degp = _deg_kernel(edge_index).reshape(NC, NDEG)

    grid = pl.cdiv(N, BN)

    g1, dinv = pl.pallas_call(
        _mm1_body,
        grid=(grid,),
        in_specs=[_row_spec(D_IN), _full_spec(D_IN, D_HID), _deg_spec()],
        out_specs=[_row_spec(D_HID), _col_spec()],
        out_shape=[jax.ShapeDtypeStruct((N, D_HID), jnp.float32),
                   jax.ShapeDtypeStruct((N, 1), jnp.float32)],
    )(x, W1, degp)

    s_pair = _scatter_kernel(g1, edge_index).reshape(NC, N, D_HID)

    g2 = pl.pallas_call(
        _mm2_body,
        grid=(grid,),
        in_specs=[_pair_spec(D_HID), _row_spec(D_HID), _col_spec(),
                  _vec_spec(D_HID), _full_spec(D_HID, D_OUT),
                  _full_spec(D_HID, D_OUT)],
        out_specs=_row_spec(D_HID),
        out_shape=jax.ShapeDtypeStruct((N, D_HID), jnp.float32),
    )(s_pair, g1, dinv, b1, Wmu, Wls)

    t_pair = _scatter_kernel(g2, edge_index).reshape(NC, N, D_HID)

    mu, logstd = pl.pallas_call(
        _fin_body,
        grid=(grid,),
        in_specs=[_pair_spec(D_HID), _row_spec(D_HID), _col_spec(),
                  _vec_spec(D_OUT), _vec_spec(D_OUT)],
        out_specs=[_row_spec(D_OUT), _row_spec(D_OUT)],
        out_shape=[jax.ShapeDtypeStruct((N, D_OUT), jnp.float32),
                   jax.ShapeDtypeStruct((N, D_OUT), jnp.float32)],
    )(t_pair, g2, dinv, bmu, bls)

    return (mu, logstd)


# speedup vs baseline: 4.2761x; 1.0321x over previous
import functools

import jax
import jax.numpy as jnp
from jax import lax
from jax.experimental import pallas as pl
from jax.experimental.pallas import tpu as pltpu
from jax.experimental.pallas import tpu_sc as plsc

N = 10000
E = 160000
D_IN = 256
D_HID = 128
D_OUT = 64

NDEG = 10240
NC = 2
NS = 16
CH = 128
EC = E // NC
CPC = EC // CH
NCHT = CPC // NS
IDXL = (NCHT + 1) * CH
RPTD = NDEG // NS
RPT = 624
RPT_LAST = 640
NB = 2

CH2 = 32
CPC2 = EC // CH2
NCHT2 = CPC2 // NS
NXTRA = CPC2 - NS * NCHT2
IDXL2 = (NCHT2 + 1) * CH2
NB2 = 8
NQ2 = 19

_mesh = plsc.VectorSubcoreMesh(core_axis_name="c", subcore_axis_name="s")


@functools.partial(
    pl.kernel,
    mesh=_mesh,
    out_type=jax.ShapeDtypeStruct((NC * NDEG,), jnp.float32),
    scratch_types=[
        pltpu.VMEM((IDXL,), jnp.int32),
        pltpu.VMEM((CH,), jnp.float32),
        pltpu.VMEM((RPTD,), jnp.float32),
        pltpu.VMEM_SHARED((NDEG,), jnp.float32),
        pltpu.SemaphoreType.DMA,
        pltpu.SemaphoreType.DMA,
    ],
)
def _deg_kernel(ei_hbm, out_hbm, idx, ones_v, zero_v, acc, d0, d1):
    c = lax.axis_index("c")
    s = lax.axis_index("s")
    e0 = (c * CPC + s * NCHT) * CH
    rbase = s * RPTD
    idx_cp = pltpu.async_copy(ei_hbm.at[1, pl.ds(e0, NCHT * CH)],
                              idx.at[pl.ds(0, NCHT * CH)], d0)
    for i in range(RPTD // 16):
        zero_v[pl.ds(i * 16, 16)] = jnp.zeros((16,), jnp.float32)
    for i in range(CH // 16):
        ones_v[pl.ds(i * 16, 16)] = jnp.ones((16,), jnp.float32)
    pltpu.sync_copy(zero_v, acc.at[pl.ds(rbase, RPTD)])
    idx_cp.wait()

    @pl.when(s == 0)
    def _():
        pltpu.sync_copy(ei_hbm.at[1, pl.ds((c * CPC + NS * NCHT) * CH, CH)],
                        idx.at[pl.ds(NCHT * CH, CH)])

    plsc.subcore_barrier()

    def sstart(ci, sem):
        pltpu.async_copy(ones_v, acc.at[idx.at[pl.ds(ci * CH, CH)]],
                         sem, add=True)

    def swait(sem):
        pltpu.make_async_copy(ones_v, acc.at[idx.at[pl.ds(0, CH)]],
                              sem).wait()

    sstart(0, d0)
    sstart(1, d1)

    def body(j, carry):
        swait(d0)
        sstart(2 * j + 2, d0)
        swait(d1)
        sstart(2 * j + 3, d1)
        return carry

    lax.fori_loop(0, NCHT // 2 - 1, body, 0)
    swait(d0)
    sstart(NCHT - 1, d0)

    @pl.when(s == 0)
    def _():
        swait(d1)
        sstart(NCHT, d1)
        swait(d1)

    @pl.when(s != 0)
    def _():
        swait(d1)

    swait(d0)
    plsc.subcore_barrier()
    pltpu.sync_copy(acc.at[pl.ds(rbase, RPTD)],
                    out_hbm.at[pl.ds(c * NDEG + rbase, RPTD)])


@functools.partial(
    pl.kernel,
    mesh=_mesh,
    out_type=jax.ShapeDtypeStruct((NC * N, D_HID), jnp.float32),
    scratch_types=[
        pltpu.VMEM((IDXL2,), jnp.int32),
        pltpu.VMEM((IDXL2,), jnp.int32),
    ] + [pltpu.VMEM((CH2, D_HID), jnp.float32)] * NB2 + [
        pltpu.VMEM_SHARED((N, D_HID), jnp.float32),
    ] + [pltpu.SemaphoreType.DMA] * (2 * NB2),
)
def _scatter_kernel(g_hbm, ei_hbm, out_hbm, idxs, idxd, *rest):
    rows = rest[:NB2]
    acc = rest[NB2]
    sg = rest[NB2 + 1:2 * NB2 + 1]
    ss = rest[2 * NB2 + 1:]
    sg0, sg1 = sg[0], sg[1]
    c = lax.axis_index("c")
    s = lax.axis_index("s")
    e0 = (c * CPC2 + s * NCHT2) * CH2
    rbase = s * RPT
    icp0 = pltpu.async_copy(ei_hbm.at[0, pl.ds(e0, NCHT2 * CH2)],
                            idxs.at[pl.ds(0, NCHT2 * CH2)], sg0)
    icp1 = pltpu.async_copy(ei_hbm.at[1, pl.ds(e0, NCHT2 * CH2)],
                            idxd.at[pl.ds(0, NCHT2 * CH2)], sg1)
    icp0.wait()
    icp1.wait()

    @pl.when(s < NXTRA)
    def _():
        ex0 = (c * CPC2 + NS * NCHT2 + s) * CH2
        pltpu.sync_copy(ei_hbm.at[0, pl.ds(ex0, CH2)],
                        idxs.at[pl.ds(NCHT2 * CH2, CH2)])
        pltpu.sync_copy(ei_hbm.at[1, pl.ds(ex0, CH2)],
                        idxd.at[pl.ds(NCHT2 * CH2, CH2)])

    def gstart(b, ci):
        pltpu.async_copy(g_hbm.at[idxs.at[pl.ds(ci * CH2, CH2)]], rows[b],
                         sg[b])

    def gwait(b):
        pltpu.make_async_copy(g_hbm.at[idxs.at[pl.ds(0, CH2)]], rows[b],
                              sg[b]).wait()

    def sstart(b, ci):
        pltpu.async_copy(rows[b], acc.at[idxd.at[pl.ds(ci * CH2, CH2)]],
                         ss[b], add=True)

    def swait(b):
        pltpu.make_async_copy(rows[b], acc.at[idxd.at[pl.ds(0, CH2)]],
                              ss[b]).wait()

    for b in range(NB2):
        gstart(b, b)

    @pl.when(s < NS - 1)
    def _():
        pltpu.sync_copy(g_hbm.at[pl.ds(rbase, RPT)],
                        acc.at[pl.ds(rbase, RPT)])

    @pl.when(s == NS - 1)
    def _():
        pltpu.sync_copy(g_hbm.at[pl.ds(rbase, RPT_LAST)],
                        acc.at[pl.ds(rbase, RPT_LAST)])

    plsc.subcore_barrier()

    NREM = NCHT2 - NB2 * NQ2

    def body(j, carry):
        base = NB2 * j
        for b in range(NB2):
            gwait(b)
            sstart(b, base + b)

        @pl.when(j < NQ2 - 1)
        def _():
            for b in range(NB2):
                swait(b)
                gstart(b, base + NB2 + b)

        return carry

    lax.fori_loop(0, NQ2, body, 0)
    for b in range(NREM):
        swait(b)
        gstart(b, NB2 * NQ2 + b)

    @pl.when(s < NXTRA)
    def _():
        swait(NREM)
        gstart(NREM, NCHT2)
        gwait(NREM)
        sstart(NREM, NCHT2)
        swait(NREM)

    @pl.when(s >= NXTRA)
    def _():
        swait(NREM)

    for b in range(NREM):
        gwait(b)
        sstart(b, NB2 * NQ2 + b)
    for b in range(NREM + 1, NB2):
        swait(b)
    for b in range(NREM):
        swait(b)
    plsc.subcore_barrier()

    @pl.when(s < NS - 1)
    def _():
        pltpu.sync_copy(acc.at[pl.ds(rbase, RPT)],
                        out_hbm.at[pl.ds(c * N + rbase, RPT)])

    @pl.when(s == NS - 1)
    def _():
        pltpu.sync_copy(acc.at[pl.ds(rbase, RPT_LAST)],
                        out_hbm.at[pl.ds(c * N + rbase, RPT_LAST)])


BN = 2048


def _mm1_body(x_ref, w_ref, p_ref, g_ref, dinv_ref):
    d = lax.rsqrt(1.0 + p_ref[0] + p_ref[1])[:, None]
    h = jnp.dot(x_ref[...], w_ref[...], preferred_element_type=jnp.float32)
    g_ref[...] = h * d
    dinv_ref[...] = d


def _mm2_body(s_ref, g1_ref, dinv_ref, b1_ref, wmu_ref, wls_ref, g2_ref):
    agg = s_ref[0] + s_ref[1] - g1_ref[...]
    h = jnp.maximum(dinv_ref[...] * agg + b1_ref[...], 0.0)
    dinv = dinv_ref[...]
    g2_ref[:, :D_OUT] = dinv * jnp.dot(
        h, wmu_ref[...], preferred_element_type=jnp.float32)
    g2_ref[:, D_OUT:] = dinv * jnp.dot(
        h, wls_ref[...], preferred_element_type=jnp.float32)


def _fin_body(t_ref, g2_ref, dinv_ref, bmu_ref, bls_ref, mu_ref, ls_ref):
    agg = t_ref[0] + t_ref[1] - g2_ref[...]
    o = dinv_ref[...] * agg
    mu_ref[...] = o[:, :D_OUT] + bmu_ref[...]
    ls_ref[...] = o[:, D_OUT:] + bls_ref[...]


def _col_spec():
    return pl.BlockSpec((BN, 1), lambda i: (i, 0))


def _row_spec(d):
    return pl.BlockSpec((BN, d), lambda i: (i, 0))


def _deg_spec():
    return pl.BlockSpec((NC, BN), lambda i: (0, i))


def _full_spec(r, d):
    return pl.BlockSpec((r, d), lambda i: (0, 0))


def _vec_spec(d):
    return pl.BlockSpec((d,), lambda i: (0,))


def _pair_spec(d):
    return pl.BlockSpec((NC, BN, d), lambda i: (0, i, 0))


def kernel(x, edge_index, W1, b1, Wmu, bmu, Wls, bls):
    degp = _deg_kernel(edge_index).reshape(NC, NDEG)

    grid = pl.cdiv(N, BN)

    g1, dinv = pl.pallas_call(
        _mm1_body,
        grid=(grid,),
        in_specs=[_row_spec(D_IN), _full_spec(D_IN, D_HID), _deg_spec()],
        out_specs=[_row_spec(D_HID), _col_spec()],
        out_shape=[jax.ShapeDtypeStruct((N, D_HID), jnp.float32),
                   jax.ShapeDtypeStruct((N, 1), jnp.float32)],
    )(x, W1, degp)

    s_pair = _scatter_kernel(g1, edge_index).reshape(NC, N, D_HID)

    g2 = pl.pallas_call(
        _mm2_body,
        grid=(grid,),
        in_specs=[_pair_spec(D_HID), _row_spec(D_HID), _col_spec(),
                  _vec_spec(D_HID), _full_spec(D_HID, D_OUT),
                  _full_spec(D_HID, D_OUT)],
        out_specs=_row_spec(D_HID),
        out_shape=jax.ShapeDtypeStruct((N, D_HID), jnp.float32),
    )(s_pair, g1, dinv, b1, Wmu, Wls)

    t_pair = _scatter_kernel(g2, edge_index).reshape(NC, N, D_HID)

    mu, logstd = pl.pallas_call(
        _fin_body,
        grid=(grid,),
        in_specs=[_pair_spec(D_HID), _row_spec(D_HID), _col_spec(),
                  _vec_spec(D_OUT), _vec_spec(D_OUT)],
        out_specs=[_row_spec(D_OUT), _row_spec(D_OUT)],
        out_shape=[jax.ShapeDtypeStruct((N, D_OUT), jnp.float32),
                   jax.ShapeDtypeStruct((N, D_OUT), jnp.float32)],
    )(t_pair, g2, dinv, bmu, bls)

    return (mu, logstd)
